# baseline (speedup 1.0000x reference)
import functools

import jax
import jax.numpy as jnp
from jax import lax
from jax.experimental import pallas as pl
from jax.experimental.pallas import tpu as pltpu
from jax.experimental.pallas import tpu_sc as plsc

_G = 16


@functools.cache
def _build(B, V, D, NC, NS):
    NW = NC * NS
    n = B // NW
    NGRP = n // _G

    mesh = plsc.VectorSubcoreMesh(core_axis_name="c", subcore_axis_name="s")

    @functools.partial(
        pl.kernel,
        mesh=mesh,
        out_type=jax.ShapeDtypeStruct((B, D), jnp.float32),
        scratch_types=[
            pltpu.VMEM((NGRP, _G), jnp.int32),
            pltpu.VMEM((2, _G, 8, D), jnp.float32),
            pltpu.VMEM((n, D), jnp.float32),
            pltpu.SemaphoreType.DMA,
            pltpu.SemaphoreType.DMA,
        ],
        compiler_params=pltpu.CompilerParams(
            use_tc_tiling_on_sc=False,
            skip_device_barrier=True,
            needs_layout_passes=False,
        ),
    )
    def gather_kernel(users_hbm, table_hbm, out_hbm, uv, tiles_v, rows_v,
                      sem_a, sem_b):
        wid = lax.axis_index("s") * NC + lax.axis_index("c")
        pltpu.sync_copy(users_hbm.at[wid], uv)
        lanes = lax.broadcasted_iota(jnp.int32, (_G,), 0)
        buf_a, buf_b = tiles_v.at[0], tiles_v.at[1]

        def fire(g, buf, sem):
            vec = uv[g, :]
            for q in range(_G):
                u = jnp.max(jnp.where(lanes == q, vec, 0))
                t = lax.shift_right_logical(u, 3)
                pltpu.async_copy(
                    table_hbm.at[pl.ds(t, 1)], buf.at[pl.ds(q, 1)], sem
                )

        def drain(buf, sem):
            pltpu.make_async_copy(
                table_hbm.at[pl.ds(0, _G)], buf, sem
            ).wait()

        def select(g, buf):
            vec = uv[g, :]
            for q in range(_G):
                u = jnp.max(jnp.where(lanes == q, vec, 0))
                r = lax.bitwise_and(u, 7)
                for m in range(D // _G):
                    rows_v[g * _G + q, pl.ds(m * _G, _G)] = buf[
                        q, r, pl.ds(m * _G, _G)
                    ]

        fire(0, buf_a, sem_a)

        def pair_body(j, carry):
            g = 2 * j
            fire(g + 1, buf_b, sem_b)
            drain(buf_a, sem_a)
            select(g, buf_a)
            fire(g + 2, buf_a, sem_a)
            drain(buf_b, sem_b)
            select(g + 1, buf_b)
            return carry

        lax.fori_loop(0, NGRP // 2 - 1, pair_body, 0)
        g_last = NGRP - 2
        fire(g_last + 1, buf_b, sem_b)
        drain(buf_a, sem_a)
        select(g_last, buf_a)
        drain(buf_b, sem_b)
        select(g_last + 1, buf_b)

        pltpu.sync_copy(rows_v, out_hbm.at[pl.ds(wid * n, n)])

    return gather_kernel


def kernel(users, user_embed):
    B, = users.shape
    V, D = user_embed.shape
    info = plsc.get_sparse_core_info()
    NC, NS = info.num_cores, info.num_subcores
    NW = NC * NS
    n = B // NW
    table3 = user_embed.reshape(V // 8, 8, D)
    u = users.astype(jnp.int32).reshape(NW, n // _G, _G)
    return _build(B, V, D, NC, NS)(u, table3)


# speedup vs baseline: 2.3477x; 2.3477x over previous
import functools

import jax
import jax.numpy as jnp
from jax import lax
from jax.experimental import pallas as pl
from jax.experimental.pallas import tpu as pltpu
from jax.experimental.pallas import tpu_sc as plsc

_G = 16


@functools.cache
def _build(B, V, D, NC, NS):
    NW = NC * NS
    n = B // NW
    NGRP = n // _G

    mesh = plsc.VectorSubcoreMesh(core_axis_name="c", subcore_axis_name="s")

    @functools.partial(
        pl.kernel,
        mesh=mesh,
        out_type=jax.ShapeDtypeStruct((B, D), jnp.float32),
        scratch_types=[
            pltpu.VMEM((NGRP, _G), jnp.int32),
            pltpu.VMEM((2, _G, 8, D), jnp.float32),
            pltpu.VMEM((n, D), jnp.float32),
            pltpu.SemaphoreType.DMA,
            pltpu.SemaphoreType.DMA,
        ],
        compiler_params=pltpu.CompilerParams(
            use_tc_tiling_on_sc=True,
            skip_device_barrier=True,
            needs_layout_passes=False,
        ),
    )
    def gather_kernel(users_hbm, table_hbm, out_hbm, uv, tiles_v, rows_v,
                      sem_a, sem_b):
        wid = lax.axis_index("s") * NC + lax.axis_index("c")
        pltpu.sync_copy(users_hbm.at[wid], uv)
        lanes = lax.broadcasted_iota(jnp.int32, (_G,), 0)
        buf_a, buf_b = tiles_v.at[0], tiles_v.at[1]

        def fire(g, buf, sem):
            vec = uv[g, :]
            for q in range(_G):
                u = jnp.max(jnp.where(lanes == q, vec, 0))
                t = lax.shift_right_logical(u, 3)
                pltpu.async_copy(
                    table_hbm.at[pl.ds(t, 1)], buf.at[pl.ds(q, 1)], sem
                )

        def drain(buf, sem):
            pltpu.make_async_copy(
                table_hbm.at[pl.ds(0, _G)], buf, sem
            ).wait()

        def select(g, buf):
            vec = uv[g, :]
            for q in range(_G):
                u = jnp.max(jnp.where(lanes == q, vec, 0))
                r = lax.bitwise_and(u, 7)
                for m in range(D // _G):
                    rows_v[g * _G + q, pl.ds(m * _G, _G)] = buf[
                        q, r, pl.ds(m * _G, _G)
                    ]

        fire(0, buf_a, sem_a)

        def pair_body(j, carry):
            g = 2 * j
            fire(g + 1, buf_b, sem_b)
            drain(buf_a, sem_a)
            select(g, buf_a)
            fire(g + 2, buf_a, sem_a)
            drain(buf_b, sem_b)
            select(g + 1, buf_b)
            return carry

        lax.fori_loop(0, NGRP // 2 - 1, pair_body, 0)
        g_last = NGRP - 2
        fire(g_last + 1, buf_b, sem_b)
        drain(buf_a, sem_a)
        select(g_last, buf_a)
        drain(buf_b, sem_b)
        select(g_last + 1, buf_b)

        pltpu.sync_copy(rows_v, out_hbm.at[pl.ds(wid * n, n)])

    return gather_kernel


def kernel(users, user_embed):
    B, = users.shape
    V, D = user_embed.shape
    info = plsc.get_sparse_core_info()
    NC, NS = info.num_cores, info.num_subcores
    NW = NC * NS
    n = B // NW
    table3 = user_embed.reshape(V // 8, 8, D)
    u = users.astype(jnp.int32).reshape(NW, n // _G, _G)
    return _build(B, V, D, NC, NS)(u, table3)
